# Initial kernel scaffold; baseline (speedup 1.0000x reference)
import jax
import jax.numpy as jnp
from jax.experimental import pallas as pl


def kernel(obs, float_metadata, pix, local_channel, local_platform, obs_type, offsets, npix, embed_tables, w1, b1, ln_g, ln_b, w2, b2, proj_w, proj_b):
    raise NotImplementedError("write your pallas kernel here")


# speedup vs baseline: 4.1590x; 4.1590x over previous
import functools

import jax
import jax.numpy as jnp
from jax import lax
from jax.experimental import pallas as pl
from jax.experimental.pallas import tpu as pltpu
from jax.experimental.pallas import tpu_sc as plsc

S = 4
NWIN = 128
PW = 8192
NPIX = 2048
NOBS = NWIN * PW
TOK = 32
N_EMBED = 1024
ED = 4
EDP = 8
META = 28
HID = 64
MOUT = 31
ODIM = 512


def _emb_gather_sc(gtype2d, table2d):
    mesh = plsc.VectorSubcoreMesh(core_axis_name="c", subcore_axis_name="s")

    @functools.partial(
        pl.kernel,
        out_type=jax.ShapeDtypeStruct((NOBS, EDP), jnp.float32),
        mesh=mesh,
        scratch_types=[
            pltpu.VMEM_SHARED((S * N_EMBED, EDP), jnp.float32),
            pltpu.VMEM((16, 128), jnp.int32),
            pltpu.VMEM((2048, EDP), jnp.float32),
            pltpu.SemaphoreType.DMA,
        ],
        compiler_params=pltpu.CompilerParams(use_tc_tiling_on_sc=False),
    )
    def k(gtype_hbm, table_hbm, emb_hbm, table_sh, idx_v, rows_v, sem):
        c = lax.axis_index("c")
        t = lax.axis_index("s")
        wid = c * 16 + t

        @pl.when(t == 0)
        def _():
            pltpu.sync_copy(table_hbm, table_sh)

        plsc.subcore_barrier()

        def chunk(kk, carry):
            r0 = wid * 256 + kk * 16
            pltpu.sync_copy(gtype_hbm.at[pl.ds(r0, 16), :], idx_v)
            descs = []
            for j in range(16):
                descs.append(
                    pltpu.async_copy(
                        table_sh.at[idx_v.at[j]],
                        rows_v.at[pl.ds(j * 128, 128)],
                        sem,
                    )
                )
            for d in descs:
                d.wait()
            pltpu.sync_copy(
                rows_v, emb_hbm.at[pl.ds(wid * 32768 + kk * 2048, 2048)]
            )
            return carry

        lax.fori_loop(0, 16, chunk, 0)

    return k(gtype2d, table2d)


def _tokenize_tc(obs2, meta3, emb3, w1, b1, ln_g, ln_b, w2, b2):
    def body(obs_ref, meta_ref, emb_ref, w1_ref, b1_ref, g_ref, bt_ref,
             w2_ref, b2_ref, out_ref):
        ob = obs_ref[0, 0]
        x = jnp.concatenate([ob[:, None], meta_ref[0], emb_ref[0][:, :ED]], axis=-1)
        h = lax.dot_general(
            x, w1_ref[0], (((1,), (1,)), ((), ())),
            preferred_element_type=jnp.float32,
        ) + b1_ref[0, 0][None, :]
        mu = jnp.mean(h, axis=-1, keepdims=True)
        var = jnp.mean((h - mu) ** 2, axis=-1, keepdims=True)
        h = (h - mu) * lax.rsqrt(var + 1e-5) * g_ref[0, 0][None, :] + bt_ref[0, 0][None, :]
        h = h * jax.nn.sigmoid(h)
        m = lax.dot_general(
            h, w2_ref[0], (((1,), (1,)), ((), ())),
            preferred_element_type=jnp.float32,
        ) + b2_ref[0, 0][None, :]
        out_ref[0] = jnp.concatenate([ob[:, None], m], axis=-1)

    sensor = lambda g: (g // (NWIN // S), 0, 0)
    return pl.pallas_call(
        body,
        grid=(NWIN,),
        in_specs=[
            pl.BlockSpec((1, 1, PW), lambda g: (g, 0, 0)),
            pl.BlockSpec((1, PW, META), lambda g: (g, 0, 0)),
            pl.BlockSpec((1, PW, EDP), lambda g: (g, 0, 0)),
            pl.BlockSpec((1, HID, 1 + META + ED), sensor),
            pl.BlockSpec((1, 1, HID), sensor),
            pl.BlockSpec((1, 1, HID), sensor),
            pl.BlockSpec((1, 1, HID), sensor),
            pl.BlockSpec((1, MOUT, HID), sensor),
            pl.BlockSpec((1, 1, MOUT), sensor),
        ],
        out_specs=pl.BlockSpec((1, PW, TOK), lambda g: (g, 0, 0)),
        out_shape=jax.ShapeDtypeStruct((NWIN, PW, TOK), jnp.float32),
    )(obs2, meta3, emb3, w1, b1, ln_g, ln_b, w2, b2)


def _scatter_sc(tok3, sidx3, zsum, zcnt, ones1):
    mesh = plsc.VectorSubcoreMesh(core_axis_name="c", subcore_axis_name="s")

    @functools.partial(
        pl.kernel,
        out_type=(
            jax.ShapeDtypeStruct((NWIN, NPIX, TOK), jnp.float32),
            jax.ShapeDtypeStruct((NWIN, NPIX, 8), jnp.float32),
        ),
        mesh=mesh,
        scratch_types=[
            pltpu.VMEM_SHARED((8 * NPIX, TOK), jnp.float32),
            pltpu.VMEM_SHARED((8 * NPIX, 8), jnp.float32),
            pltpu.VMEM((16, 128), jnp.int32),
            pltpu.VMEM((2048, TOK), jnp.float32),
            pltpu.VMEM((128, 8), jnp.float32),
            pltpu.SemaphoreType.DMA,
        ],
        compiler_params=pltpu.CompilerParams(use_tc_tiling_on_sc=False),
    )
    def k(tok_hbm, sidx_hbm, zsum_hbm, zcnt_hbm, ones_hbm,
          bsum_hbm, bcnt_hbm, sh_sum, sh_cnt, idx_v, tok_v, ones_v, sem):
        c = lax.axis_index("c")
        t = lax.axis_index("s")
        k_slab = t // 2
        half = t % 2
        slab = k_slab * NPIX
        pltpu.sync_copy(ones_hbm, ones_v)

        def rnd(r, carry):
            win = r * 16 + c * 8 + k_slab

            @pl.when(half == 0)
            def _():
                pltpu.sync_copy(zsum_hbm, sh_sum.at[pl.ds(slab, NPIX)])
                pltpu.sync_copy(zcnt_hbm, sh_cnt.at[pl.ds(slab, NPIX)])

            plsc.subcore_barrier()

            def chunk(kk, carry2):
                row0 = half * 32 + kk * 16
                pltpu.sync_copy(sidx_hbm.at[win, pl.ds(row0, 16), :], idx_v)
                pltpu.sync_copy(
                    tok_hbm.at[win, pl.ds(row0 * 128, 2048), :], tok_v
                )
                descs = []
                for j in range(16):
                    descs.append(
                        pltpu.async_copy(
                            tok_v.at[pl.ds(j * 128, 128)],
                            sh_sum.at[idx_v.at[j]],
                            sem, add=True,
                        )
                    )
                    descs.append(
                        pltpu.async_copy(
                            ones_v, sh_cnt.at[idx_v.at[j]], sem, add=True,
                        )
                    )
                for d in descs:
                    d.wait()
                return carry2

            lax.fori_loop(0, PW // 2 // 2048, chunk, 0)
            plsc.subcore_barrier()

            @pl.when(half == 0)
            def _():
                pltpu.sync_copy(sh_sum.at[pl.ds(slab, NPIX)], bsum_hbm.at[win])

            @pl.when(half == 1)
            def _():
                pltpu.sync_copy(sh_cnt.at[pl.ds(slab, NPIX)], bcnt_hbm.at[win])

            plsc.subcore_barrier()
            return carry

        lax.fori_loop(0, NWIN // 16, rnd, 0)

    return k(tok3, sidx3, zsum, zcnt, ones1)


def _project_tc(bsum4, bcnt4, projcat, proj_b):
    def body(s_ref, c_ref, pw_ref, pb_ref, out_ref):
        ms = []
        for s in range(S):
            cnt = jnp.maximum(c_ref[s, 0][:, :1], 1.0)
            ms.append(s_ref[s, 0] / cnt)
        mc = jnp.concatenate(ms, axis=-1)
        acc = jnp.dot(mc, pw_ref[...], preferred_element_type=jnp.float32)
        out_ref[0] = (acc + jnp.sum(pb_ref[...], axis=0)[None, :]) * (1.0 / S)

    return pl.pallas_call(
        body,
        grid=(NWIN // S,),
        in_specs=[
            pl.BlockSpec((S, 1, NPIX, TOK), lambda g: (0, g, 0, 0)),
            pl.BlockSpec((S, 1, NPIX, 8), lambda g: (0, g, 0, 0)),
            pl.BlockSpec((S * TOK, ODIM), lambda g: (0, 0)),
            pl.BlockSpec((S, ODIM), lambda g: (0, 0)),
        ],
        out_specs=pl.BlockSpec((1, NPIX, ODIM), lambda g: (g, 0, 0)),
        out_shape=jax.ShapeDtypeStruct((NWIN // S, NPIX, ODIM), jnp.float32),
    )(bsum4, bcnt4, projcat, proj_b)


def kernel(obs, float_metadata, pix, local_channel, local_platform, obs_type,
           offsets, npix, embed_tables, w1, b1, ln_g, ln_b, w2, b2,
           proj_w, proj_b):
    del local_channel, local_platform, offsets, npix
    gtype = (
        obs_type.astype(jnp.int32).reshape(S, NOBS // S)
        + (jnp.arange(S, dtype=jnp.int32) * N_EMBED)[:, None]
    ).reshape(8192, 128)
    table2 = jnp.pad(embed_tables.reshape(S * N_EMBED, ED), ((0, 0), (0, EDP - ED)))
    emb = _emb_gather_sc(gtype, table2)

    tok = _tokenize_tc(
        obs.reshape(NWIN, 1, PW),
        float_metadata.reshape(NWIN, PW, META),
        emb.reshape(NWIN, PW, EDP),
        w1, b1.reshape(S, 1, HID), ln_g.reshape(S, 1, HID),
        ln_b.reshape(S, 1, HID), w2, b2.reshape(S, 1, MOUT),
    )

    winmod = (jnp.arange(NWIN, dtype=jnp.int32) % 8) * NPIX
    sidx = (pix.astype(jnp.int32).reshape(NWIN, PW) + winmod[:, None]).reshape(
        NWIN, PW // 128, 128
    )
    zsum = jnp.zeros((NPIX, TOK), jnp.float32)
    zcnt = jnp.zeros((NPIX, 8), jnp.float32)
    ones1 = jnp.ones((128, 8), jnp.float32)
    bsum, bcnt = _scatter_sc(tok, sidx, zsum, zcnt, ones1)

    projcat = jnp.swapaxes(proj_w, 1, 2).reshape(S * TOK, ODIM)
    out = _project_tc(
        bsum.reshape(S, NWIN // S, NPIX, TOK),
        bcnt.reshape(S, NWIN // S, NPIX, 8),
        projcat, proj_b,
    )
    return out.reshape(4, 8, NPIX, ODIM)


# speedup vs baseline: 4.2864x; 1.0306x over previous
import functools

import jax
import jax.numpy as jnp
from jax import lax
from jax.experimental import pallas as pl
from jax.experimental.pallas import tpu as pltpu
from jax.experimental.pallas import tpu_sc as plsc

S = 4
NWIN = 128
PW = 8192
NPIX = 2048
NOBS = NWIN * PW
TOK = 32
N_EMBED = 1024
ED = 4
EDP = 8
META = 28
HID = 64
MOUT = 31
ODIM = 512


def _emb_gather_sc(gtype2d, table2d):
    mesh = plsc.VectorSubcoreMesh(core_axis_name="c", subcore_axis_name="s")

    @functools.partial(
        pl.kernel,
        out_type=jax.ShapeDtypeStruct((NOBS, EDP), jnp.float32),
        mesh=mesh,
        scratch_types=[
            pltpu.VMEM_SHARED((S * N_EMBED, EDP), jnp.float32),
            pltpu.VMEM((16, 128), jnp.int32),
            pltpu.VMEM((2048, EDP), jnp.float32),
            pltpu.SemaphoreType.DMA,
        ],
        compiler_params=pltpu.CompilerParams(use_tc_tiling_on_sc=False),
    )
    def k(gtype_hbm, table_hbm, emb_hbm, table_sh, idx_v, rows_v, sem):
        c = lax.axis_index("c")
        t = lax.axis_index("s")
        wid = c * 16 + t

        @pl.when(t == 0)
        def _():
            pltpu.sync_copy(table_hbm, table_sh)

        plsc.subcore_barrier()

        def chunk(kk, carry):
            r0 = wid * 256 + kk * 16
            pltpu.sync_copy(gtype_hbm.at[pl.ds(r0, 16), :], idx_v)
            descs = []
            for j in range(16):
                descs.append(
                    pltpu.async_copy(
                        table_sh.at[idx_v.at[j]],
                        rows_v.at[pl.ds(j * 128, 128)],
                        sem,
                    )
                )
            for d in descs:
                d.wait()
            pltpu.sync_copy(
                rows_v, emb_hbm.at[pl.ds(wid * 32768 + kk * 2048, 2048)]
            )
            return carry

        lax.fori_loop(0, 16, chunk, 0)

    return k(gtype2d, table2d)


def _tokenize_tc(obs1, metaT, emb3, w1mT, w1o, w1e, b1, ln_g, ln_b, w2, b2):
    def body(obs_ref, meta_ref, emb_ref, w1m_ref, w1o_ref, w1e_ref, b1_ref,
             g_ref, bt_ref, w2_ref, b2_ref, out_ref):
        ob = obs_ref[...]
        mT = meta_ref[...]
        ee = emb_ref[0]
        embl = jnp.concatenate(
            [ee[:, 8 * j:8 * j + 8] for j in range(16)], axis=0)
        h = lax.dot_general(
            mT, w1m_ref[0], (((0,), (0,)), ((), ())),
            preferred_element_type=jnp.float32,
        )
        h = h + lax.dot_general(
            embl, w1e_ref[0], (((1,), (0,)), ((), ())),
            preferred_element_type=jnp.float32,
        )
        h = h + ob[:, None] * w1o_ref[0, 0][None, :] + b1_ref[0, 0][None, :]
        mu = jnp.mean(h, axis=-1, keepdims=True)
        var = jnp.mean((h - mu) ** 2, axis=-1, keepdims=True)
        h = (h - mu) * lax.rsqrt(var + 1e-5) * g_ref[0, 0][None, :] + bt_ref[0, 0][None, :]
        h = h * jax.nn.sigmoid(h)
        m = lax.dot_general(
            h, w2_ref[0], (((1,), (1,)), ((), ())),
            preferred_element_type=jnp.float32,
        ) + b2_ref[0, 0][None, :]
        tok = jnp.concatenate([ob[:, None], m], axis=-1)
        out_ref[0] = jnp.concatenate(
            [tok[2048 * q:2048 * (q + 1)] for q in range(4)], axis=-1)

    sensor = lambda g: (g // (NWIN // S), 0, 0)
    return pl.pallas_call(
        body,
        grid=(NWIN,),
        in_specs=[
            pl.BlockSpec((PW,), lambda g: (g,)),
            pl.BlockSpec((META, PW), lambda g: (0, g)),
            pl.BlockSpec((1, PW // 16, 128), lambda g: (g, 0, 0)),
            pl.BlockSpec((1, META, HID), sensor),
            pl.BlockSpec((1, 1, HID), sensor),
            pl.BlockSpec((1, EDP, HID), sensor),
            pl.BlockSpec((1, 1, HID), sensor),
            pl.BlockSpec((1, 1, HID), sensor),
            pl.BlockSpec((1, 1, HID), sensor),
            pl.BlockSpec((1, MOUT, HID), sensor),
            pl.BlockSpec((1, 1, MOUT), sensor),
        ],
        out_specs=pl.BlockSpec((1, PW // 4, 128), lambda g: (g, 0, 0)),
        out_shape=jax.ShapeDtypeStruct((NWIN, PW // 4, 128), jnp.float32),
    )(obs1, metaT, emb3, w1mT, w1o, w1e, b1, ln_g, ln_b, w2, b2)


def _scatter_sc(tok3, sidx3, zsum, zcnt, ones1):
    mesh = plsc.VectorSubcoreMesh(core_axis_name="c", subcore_axis_name="s")

    @functools.partial(
        pl.kernel,
        out_type=(
            jax.ShapeDtypeStruct((NWIN, NPIX // 4, 128), jnp.float32),
            jax.ShapeDtypeStruct((NWIN, NPIX // 16, 128), jnp.float32),
        ),
        mesh=mesh,
        scratch_types=[
            pltpu.VMEM_SHARED((8 * NPIX, TOK), jnp.float32),
            pltpu.VMEM_SHARED((8 * NPIX, 8), jnp.float32),
            pltpu.VMEM((16, 128), jnp.int32),
            pltpu.VMEM((2048, TOK), jnp.float32),
            pltpu.VMEM((128, 8), jnp.float32),
            pltpu.SemaphoreType.DMA,
        ],
        compiler_params=pltpu.CompilerParams(use_tc_tiling_on_sc=False),
    )
    def k(tok_hbm, sidx_hbm, zsum_hbm, zcnt_hbm, ones_hbm,
          bsum_hbm, bcnt_hbm, sh_sum, sh_cnt, idx_v, tok_v, ones_v, sem):
        c = lax.axis_index("c")
        t = lax.axis_index("s")
        k_slab = t // 2
        half = t % 2
        slab = k_slab * NPIX
        pltpu.sync_copy(ones_hbm, ones_v)

        def rnd(r, carry):
            win = r * 16 + c * 8 + k_slab

            @pl.when(half == 0)
            def _():
                pltpu.sync_copy(zsum_hbm, sh_sum.at[pl.ds(slab, NPIX)])
                pltpu.sync_copy(zcnt_hbm, sh_cnt.at[pl.ds(slab, NPIX)])

            plsc.subcore_barrier()

            def chunk(kk, carry2):
                row0 = half * 32 + kk * 16
                pltpu.sync_copy(sidx_hbm.at[win, pl.ds(row0, 16), :], idx_v)
                pltpu.sync_copy(
                    tok_hbm.at[win, pl.ds(row0 * 128, 2048), :], tok_v
                )
                descs = []
                for j in range(16):
                    descs.append(
                        pltpu.async_copy(
                            tok_v.at[pl.ds(j * 128, 128)],
                            sh_sum.at[idx_v.at[j]],
                            sem, add=True,
                        )
                    )
                    descs.append(
                        pltpu.async_copy(
                            ones_v, sh_cnt.at[idx_v.at[j]], sem, add=True,
                        )
                    )
                for d in descs:
                    d.wait()
                return carry2

            lax.fori_loop(0, PW // 2 // 2048, chunk, 0)
            plsc.subcore_barrier()

            @pl.when(half == 0)
            def _():
                for q in range(4):
                    pltpu.sync_copy(
                        sh_sum.at[pl.ds(slab + q * 512, 512)],
                        bsum_hbm.at[win, :, pl.ds(32 * q, 32)],
                    )

            @pl.when(half == 1)
            def _():
                for q in range(16):
                    pltpu.sync_copy(
                        sh_cnt.at[pl.ds(slab + q * 128, 128)],
                        bcnt_hbm.at[win, :, pl.ds(8 * q, 8)],
                    )

            plsc.subcore_barrier()
            return carry

        lax.fori_loop(0, NWIN // 16, rnd, 0)

    return k(tok3, sidx3, zsum, zcnt, ones1)


def _project_tc(bsum4, bcnt4, projcat, proj_b):
    def body(s_ref, c_ref, pw_ref, pb_ref, out_ref):
        ms = []
        for s in range(S):
            bb = s_ref[s, 0]
            cc = c_ref[s, 0]
            sums = jnp.concatenate(
                [bb[:, 32 * q:32 * (q + 1)] for q in range(4)], axis=0)
            cnt8 = jnp.concatenate(
                [cc[:, 8 * q:8 * (q + 1)] for q in range(16)], axis=0)
            cnt = jnp.maximum(cnt8[:, :1], 1.0)
            ms.append(sums / cnt)
        mc = jnp.concatenate(ms, axis=-1)
        acc = jnp.dot(mc, pw_ref[...], preferred_element_type=jnp.float32)
        out_ref[0] = (acc + jnp.sum(pb_ref[...], axis=0)[None, :]) * (1.0 / S)

    return pl.pallas_call(
        body,
        grid=(NWIN // S,),
        in_specs=[
            pl.BlockSpec((S, 1, NPIX // 4, 128), lambda g: (0, g, 0, 0)),
            pl.BlockSpec((S, 1, NPIX // 16, 128), lambda g: (0, g, 0, 0)),
            pl.BlockSpec((S * TOK, ODIM), lambda g: (0, 0)),
            pl.BlockSpec((S, ODIM), lambda g: (0, 0)),
        ],
        out_specs=pl.BlockSpec((1, NPIX, ODIM), lambda g: (g, 0, 0)),
        out_shape=jax.ShapeDtypeStruct((NWIN // S, NPIX, ODIM), jnp.float32),
    )(bsum4, bcnt4, projcat, proj_b)


def kernel(obs, float_metadata, pix, local_channel, local_platform, obs_type,
           offsets, npix, embed_tables, w1, b1, ln_g, ln_b, w2, b2,
           proj_w, proj_b):
    del local_channel, local_platform, offsets, npix
    gtype = (
        obs_type.astype(jnp.int32).reshape(S, NOBS // S)
        + (jnp.arange(S, dtype=jnp.int32) * N_EMBED)[:, None]
    ).reshape(NWIN, 16, PW // 16)
    gtype_p = jnp.swapaxes(gtype, 1, 2).reshape(8192, 128)
    table2 = jnp.pad(embed_tables.reshape(S * N_EMBED, ED), ((0, 0), (0, EDP - ED)))
    emb = _emb_gather_sc(gtype_p, table2)

    tok2 = _tokenize_tc(
        obs,
        jnp.swapaxes(float_metadata, 0, 1),
        emb.reshape(NWIN, PW // 16, 128),
        jnp.swapaxes(w1[:, :, 1:1 + META], 1, 2),
        w1[:, :, 0].reshape(S, 1, HID),
        jnp.pad(jnp.swapaxes(w1[:, :, 1 + META:], 1, 2), ((0, 0), (0, EDP - ED), (0, 0))),
        b1.reshape(S, 1, HID), ln_g.reshape(S, 1, HID), ln_b.reshape(S, 1, HID),
        w2, b2.reshape(S, 1, MOUT),
    )

    winmod = (jnp.arange(NWIN, dtype=jnp.int32) % 8) * NPIX
    sidx = (
        pix.astype(jnp.int32).reshape(NWIN, 4, PW // 4)
        + winmod[:, None, None]
    )
    sidx_p = jnp.swapaxes(sidx, 1, 2).reshape(NWIN, PW // 128, 128)
    zsum = jnp.zeros((NPIX, TOK), jnp.float32)
    zcnt = jnp.zeros((NPIX, 8), jnp.float32)
    ones1 = jnp.ones((128, 8), jnp.float32)
    bsum, bcnt = _scatter_sc(
        tok2.reshape(-1).reshape(NWIN, PW, TOK), sidx_p, zsum, zcnt, ones1)

    projcat = jnp.swapaxes(proj_w, 1, 2).reshape(S * TOK, ODIM)
    out = _project_tc(
        bsum.reshape(S, NWIN // S, NPIX // 4, 128),
        bcnt.reshape(S, NWIN // S, NPIX // 16, 128),
        projcat, proj_b,
    )
    return out.reshape(4, 8, NPIX, ODIM)


# speedup vs baseline: 5.9971x; 1.3991x over previous
import functools

import jax
import jax.numpy as jnp
from jax import lax
from jax.experimental import pallas as pl
from jax.experimental.pallas import tpu as pltpu
from jax.experimental.pallas import tpu_sc as plsc

S = 4
NWIN = 128
PW = 8192
NPIX = 2048
NOBS = NWIN * PW
TOK = 32
N_EMBED = 1024
ED = 4
EDP = 8
META = 28
HID = 64
MOUT = 31
ODIM = 512


def _emb_gather_sc(gtype2d, table2d):
    mesh = plsc.VectorSubcoreMesh(core_axis_name="c", subcore_axis_name="s")

    @functools.partial(
        pl.kernel,
        out_type=jax.ShapeDtypeStruct((NOBS, EDP), jnp.float32),
        mesh=mesh,
        scratch_types=[
            pltpu.VMEM_SHARED((S * N_EMBED, EDP), jnp.float32),
            pltpu.VMEM((16, 128), jnp.int32),
            pltpu.VMEM((2048, EDP), jnp.float32),
            pltpu.SemaphoreType.DMA,
        ],
        compiler_params=pltpu.CompilerParams(use_tc_tiling_on_sc=False),
    )
    def k(gtype_hbm, table_hbm, emb_hbm, table_sh, idx_v, rows_v, sem):
        c = lax.axis_index("c")
        t = lax.axis_index("s")
        wid = c * 16 + t

        @pl.when(t == 0)
        def _():
            pltpu.sync_copy(table_hbm, table_sh)

        plsc.subcore_barrier()

        def chunk(kk, carry):
            r0 = wid * 256 + kk * 16
            pltpu.sync_copy(gtype_hbm.at[pl.ds(r0, 16), :], idx_v)
            descs = []
            for j in range(16):
                descs.append(
                    pltpu.async_copy(
                        table_sh.at[idx_v.at[j]],
                        rows_v.at[pl.ds(j * 128, 128)],
                        sem,
                    )
                )
            for d in descs:
                d.wait()
            pltpu.sync_copy(
                rows_v, emb_hbm.at[pl.ds(wid * 32768 + kk * 2048, 2048)]
            )
            return carry

        lax.fori_loop(0, 16, chunk, 0)

    return k(gtype2d, table2d)


def _tokenize_tc(obs1, metaT, emb3, w1mT, w1o, w1e, b1, ln_g, ln_b, w2, b2):
    def body(obs_ref, meta_ref, emb_ref, w1m_ref, w1o_ref, w1e_ref, b1_ref,
             g_ref, bt_ref, w2_ref, b2_ref, out_ref):
        ob = obs_ref[...]
        mT = meta_ref[...]
        ee = emb_ref[0]
        embl = jnp.concatenate(
            [ee[:, 8 * j:8 * j + 8] for j in range(16)], axis=0)
        h = lax.dot_general(
            mT, w1m_ref[0], (((0,), (0,)), ((), ())),
            preferred_element_type=jnp.float32,
        )
        h = h + lax.dot_general(
            embl, w1e_ref[0], (((1,), (0,)), ((), ())),
            preferred_element_type=jnp.float32,
        )
        h = h + ob[:, None] * w1o_ref[0, 0][None, :] + b1_ref[0, 0][None, :]
        mu = jnp.mean(h, axis=-1, keepdims=True)
        var = jnp.mean((h - mu) ** 2, axis=-1, keepdims=True)
        h = (h - mu) * lax.rsqrt(var + 1e-5) * g_ref[0, 0][None, :] + bt_ref[0, 0][None, :]
        h = h * jax.nn.sigmoid(h)
        m = lax.dot_general(
            h, w2_ref[0], (((1,), (1,)), ((), ())),
            preferred_element_type=jnp.float32,
        ) + b2_ref[0, 0][None, :]
        tok = jnp.concatenate([ob[:, None], m], axis=-1)
        out_ref[0] = jnp.concatenate(
            [tok[2048 * q:2048 * (q + 1)] for q in range(4)], axis=-1)

    sensor = lambda g: (g // (NWIN // S), 0, 0)
    return pl.pallas_call(
        body,
        grid=(NWIN,),
        in_specs=[
            pl.BlockSpec((PW,), lambda g: (g,)),
            pl.BlockSpec((META, PW), lambda g: (0, g)),
            pl.BlockSpec((1, PW // 16, 128), lambda g: (g, 0, 0)),
            pl.BlockSpec((1, META, HID), sensor),
            pl.BlockSpec((1, 1, HID), sensor),
            pl.BlockSpec((1, EDP, HID), sensor),
            pl.BlockSpec((1, 1, HID), sensor),
            pl.BlockSpec((1, 1, HID), sensor),
            pl.BlockSpec((1, 1, HID), sensor),
            pl.BlockSpec((1, MOUT, HID), sensor),
            pl.BlockSpec((1, 1, MOUT), sensor),
        ],
        out_specs=pl.BlockSpec((1, PW // 4, 128), lambda g: (g, 0, 0)),
        out_shape=jax.ShapeDtypeStruct((NWIN, PW // 4, 128), jnp.float32),
    )(obs1, metaT, emb3, w1mT, w1o, w1e, b1, ln_g, ln_b, w2, b2)


def _scatter_sc(tok3, sidx3, zsum, zcnt, ones1):
    mesh = plsc.VectorSubcoreMesh(core_axis_name="c", subcore_axis_name="s")

    @functools.partial(
        pl.kernel,
        out_type=(
            jax.ShapeDtypeStruct((NWIN, NPIX // 4, 128), jnp.float32),
            jax.ShapeDtypeStruct((NWIN, NPIX // 16, 128), jnp.float32),
        ),
        mesh=mesh,
        scratch_types=[
            pltpu.VMEM_SHARED((8 * NPIX, TOK), jnp.float32),
            pltpu.VMEM_SHARED((8 * NPIX, 8), jnp.float32),
            pltpu.VMEM((16, 128), jnp.int32),
            pltpu.VMEM((2048, TOK), jnp.float32),
            pltpu.VMEM((128, 8), jnp.float32),
            pltpu.SemaphoreType.DMA,
        ],
        compiler_params=pltpu.CompilerParams(use_tc_tiling_on_sc=False),
    )
    def k(tok_hbm, sidx_hbm, zsum_hbm, zcnt_hbm, ones_hbm,
          bsum_hbm, bcnt_hbm, sh_sum, sh_cnt, idx_v, tok_v, ones_v, sem):
        c = lax.axis_index("c")
        t = lax.axis_index("s")
        k_slab = t // 2
        half = t % 2
        slab = k_slab * NPIX
        pltpu.sync_copy(ones_hbm, ones_v)

        def rnd(r, carry):
            win = r * 16 + c * 8 + k_slab

            @pl.when(half == 0)
            def _():
                pltpu.sync_copy(zsum_hbm, sh_sum.at[pl.ds(slab, NPIX)])
                pltpu.sync_copy(zcnt_hbm, sh_cnt.at[pl.ds(slab, NPIX)])

            plsc.subcore_barrier()

            def chunk(kk, carry2):
                q = half * 2 + kk
                pltpu.sync_copy(sidx_hbm.at[win, pl.ds(q * 16, 16), :], idx_v)
                pltpu.sync_copy(
                    tok_hbm.at[win, :, pl.ds(q * 32, TOK)], tok_v
                )
                descs = []
                for j in range(16):
                    descs.append(
                        pltpu.async_copy(
                            tok_v.at[pl.ds(j * 128, 128)],
                            sh_sum.at[idx_v.at[j]],
                            sem, add=True,
                        )
                    )
                    descs.append(
                        pltpu.async_copy(
                            ones_v, sh_cnt.at[idx_v.at[j]], sem, add=True,
                        )
                    )
                for d in descs:
                    d.wait()
                return carry2

            lax.fori_loop(0, 2, chunk, 0)
            plsc.subcore_barrier()

            @pl.when(half == 0)
            def _():
                for q in range(4):
                    pltpu.sync_copy(
                        sh_sum.at[pl.ds(slab + q * 512, 512)],
                        bsum_hbm.at[win, :, pl.ds(32 * q, 32)],
                    )

            @pl.when(half == 1)
            def _():
                for q in range(16):
                    pltpu.sync_copy(
                        sh_cnt.at[pl.ds(slab + q * 128, 128)],
                        bcnt_hbm.at[win, :, pl.ds(8 * q, 8)],
                    )

            plsc.subcore_barrier()
            return carry

        lax.fori_loop(0, NWIN // 16, rnd, 0)

    return k(tok3, sidx3, zsum, zcnt, ones1)


def _project_tc(bsum4, bcnt4, projcat, proj_b):
    def body(s_ref, c_ref, pw_ref, pb_ref, out_ref):
        ms = []
        for s in range(S):
            bb = s_ref[s, 0]
            cc = c_ref[s, 0]
            sums = jnp.concatenate(
                [bb[:, 32 * q:32 * (q + 1)] for q in range(4)], axis=0)
            cnt8 = jnp.concatenate(
                [cc[:, 8 * q:8 * (q + 1)] for q in range(16)], axis=0)
            cnt = jnp.maximum(cnt8[:, :1], 1.0)
            ms.append(sums / cnt)
        mc = jnp.concatenate(ms, axis=-1)
        acc = jnp.dot(mc, pw_ref[...], preferred_element_type=jnp.float32)
        out_ref[0] = (acc + jnp.sum(pb_ref[...], axis=0)[None, :]) * (1.0 / S)

    return pl.pallas_call(
        body,
        grid=(NWIN // S,),
        in_specs=[
            pl.BlockSpec((S, 1, NPIX // 4, 128), lambda g: (0, g, 0, 0)),
            pl.BlockSpec((S, 1, NPIX // 16, 128), lambda g: (0, g, 0, 0)),
            pl.BlockSpec((S * TOK, ODIM), lambda g: (0, 0)),
            pl.BlockSpec((S, ODIM), lambda g: (0, 0)),
        ],
        out_specs=pl.BlockSpec((1, NPIX, ODIM), lambda g: (g, 0, 0)),
        out_shape=jax.ShapeDtypeStruct((NWIN // S, NPIX, ODIM), jnp.float32),
    )(bsum4, bcnt4, projcat, proj_b)


def kernel(obs, float_metadata, pix, local_channel, local_platform, obs_type,
           offsets, npix, embed_tables, w1, b1, ln_g, ln_b, w2, b2,
           proj_w, proj_b):
    del local_channel, local_platform, offsets, npix
    gtype = (
        obs_type.astype(jnp.int32).reshape(S, NOBS // S)
        + (jnp.arange(S, dtype=jnp.int32) * N_EMBED)[:, None]
    ).reshape(NWIN, 16, PW // 16)
    gtype_p = jnp.swapaxes(gtype, 1, 2).reshape(8192, 128)
    table2 = jnp.pad(embed_tables.reshape(S * N_EMBED, ED), ((0, 0), (0, EDP - ED)))
    emb = _emb_gather_sc(gtype_p, table2)

    tok2 = _tokenize_tc(
        obs,
        jnp.swapaxes(float_metadata, 0, 1),
        emb.reshape(NWIN, PW // 16, 128),
        jnp.swapaxes(w1[:, :, 1:1 + META], 1, 2),
        w1[:, :, 0].reshape(S, 1, HID),
        jnp.pad(jnp.swapaxes(w1[:, :, 1 + META:], 1, 2), ((0, 0), (0, EDP - ED), (0, 0))),
        b1.reshape(S, 1, HID), ln_g.reshape(S, 1, HID), ln_b.reshape(S, 1, HID),
        w2, b2.reshape(S, 1, MOUT),
    )

    winmod = (jnp.arange(NWIN, dtype=jnp.int32) % 8) * NPIX
    sidx_p = (
        pix.astype(jnp.int32).reshape(NWIN, PW // 128, 128)
        + winmod[:, None, None]
    )
    zsum = jnp.zeros((NPIX, TOK), jnp.float32)
    zcnt = jnp.zeros((NPIX, 8), jnp.float32)
    ones1 = jnp.ones((128, 8), jnp.float32)
    bsum, bcnt = _scatter_sc(tok2, sidx_p, zsum, zcnt, ones1)

    projcat = jnp.swapaxes(proj_w, 1, 2).reshape(S * TOK, ODIM)
    out = _project_tc(
        bsum.reshape(S, NWIN // S, NPIX // 4, 128),
        bcnt.reshape(S, NWIN // S, NPIX // 16, 128),
        projcat, proj_b,
    )
    return out.reshape(4, 8, NPIX, ODIM)
